# Initial kernel scaffold; baseline (speedup 1.0000x reference)
import jax
import jax.numpy as jnp
from jax.experimental import pallas as pl


def kernel(x, edge_index, edge_attr, W_sg, b_sg, W_dg, b_dg, W_eg, b_eg, W_su, b_su, W_du, b_du, bn_e_g, bn_e_b, bn_n_g, bn_n_b):
    raise NotImplementedError("write your pallas kernel here")


# speedup vs baseline: 1.6730x; 1.6730x over previous
import functools

import jax
import jax.numpy as jnp
from jax import lax
from jax.experimental import pallas as pl
from jax.experimental.pallas import tpu as pltpu
from jax.experimental.pallas import tpu_sc as plsc

F32 = jnp.float32


def _pick_block(n, target):
    d = min(n, target)
    while n % d:
        d -= 1
    return d


def _softplus(z):
    return jnp.maximum(z, 0.0) + jnp.log1p(jnp.exp(-jnp.abs(z)))


def _proj_body(x_ref, w_ref, b_ref, out_ref):
    xb = x_ref[...]
    for h in range(2):
        out_ref[h] = (
            lax.dot_general(xb, w_ref[h], (((1,), (1,)), ((), ())),
                            preferred_element_type=F32)
            + b_ref[h]
        )


def _project_halved(a, W, b, block_rows):
    R, D = a.shape
    H = D // 2
    grid = (R // block_rows,)
    W2 = W.reshape(2, H, D)
    b2 = b.reshape(2, H)
    return pl.pallas_call(
        _proj_body,
        grid=grid,
        in_specs=[
            pl.BlockSpec((block_rows, D), lambda i: (i, 0)),
            pl.BlockSpec((2, H, D), lambda i: (0, 0, 0)),
            pl.BlockSpec((2, H), lambda i: (0, 0)),
        ],
        out_specs=pl.BlockSpec((2, block_rows, H), lambda i: (0, i, 0)),
        out_shape=jax.ShapeDtypeStruct((2, R, H), F32),
    )(a, W2, b2)


def _sc_edge_body(row_h, col_h, xs_h, xd_h, xu_h, ea_h,
                  m_h, hsum_h, ssum_h,
                  h_acc, s_acc, ridx, cidx, cgidx,
                  xsb, xdb, xub, eab, mb, sem,
                  *, N, E, C):
    c = lax.axis_index("c")
    s = lax.axis_index("s")
    num_chunks = E // C

    q, r = divmod(N // 8, 16)
    counts = sorted({8 * q} | ({8 * (q + 1)} if r else set()))
    base0 = 8 * (q * s + jnp.minimum(s, r))

    def _zero_row(e, _):
        for j in range(4):
            mb[e, pl.ds(j * 16, 16)] = jnp.zeros((16,), F32)
        return 0
    lax.fori_loop(0, C, _zero_row, 0)
    rows_per_sub = 8 * (q + (s < r).astype(jnp.int32))
    fl = min(120, C)
    for cnt in counts:
        if cnt == 0:
            continue

        @pl.when(rows_per_sub == cnt)
        def _(cnt=cnt):
            off = 0
            while off < cnt:
                n = min(fl, cnt - off)
                pltpu.sync_copy(mb.at[pl.ds(0, n)],
                                h_acc.at[pl.ds(base0 + off, n)])
                pltpu.sync_copy(mb.at[pl.ds(0, n)],
                                s_acc.at[pl.ds(base0 + off, n)])
                off += n
    plsc.subcore_barrier()

    tmax = (num_chunks + 15) // 16

    def _chunk(t, _):
        chunk = s + t * 16

        @pl.when(chunk < num_chunks)
        def _():
            base = chunk * C
            pltpu.sync_copy(row_h.at[pl.ds(base, C)], ridx)
            pltpu.sync_copy(col_h.at[pl.ds(base, C)], cidx)
            goff = c * N
            for j in range(C // 16):
                sl = pl.ds(j * 16, 16)
                ridx[sl] = ridx[sl] + goff
                cgidx[sl] = cidx[sl] + goff
            d1 = pltpu.async_copy(xs_h.at[ridx], xsb, sem)
            d2 = pltpu.async_copy(xd_h.at[cgidx], xdb, sem)
            d3 = pltpu.async_copy(xu_h.at[cgidx], xub, sem)
            d4 = pltpu.async_copy(ea_h.at[pl.ds(c * E + base, C)], eab, sem)
            d1.wait(); d2.wait(); d3.wait(); d4.wait()

            def _edge(e, _):
                for j in range(4):
                    sl = pl.ds(j * 16, 16)
                    mv = xsb[e, sl] + xdb[e, sl] + eab[e, sl]
                    mb[e, sl] = mv
                    sg = 1.0 / (1.0 + jnp.exp(-mv))
                    xsb[e, sl] = sg
                    xub[e, sl] = xub[e, sl] * sg
                return 0
            lax.fori_loop(0, C, _edge, 0)

            pltpu.sync_copy(mb, m_h.at[pl.ds(c * E + base, C)])
            pltpu.sync_copy(xsb, s_acc.at[cidx], add=True)
            pltpu.sync_copy(xub, h_acc.at[cidx], add=True)
        return 0

    lax.fori_loop(0, tmax, _chunk, 0)
    plsc.subcore_barrier()

    for cnt in counts:
        if cnt == 0:
            continue

        @pl.when(rows_per_sub == cnt)
        def _(cnt=cnt):
            off = 0
            while off < cnt:
                n = min(fl, cnt - off)
                src = pl.ds(base0 + off, n)
                dst = pl.ds(c * N + base0 + off, n)
                pltpu.sync_copy(h_acc.at[src], hsum_h.at[dst])
                pltpu.sync_copy(s_acc.at[src], ssum_h.at[dst])
                off += n


def _sc_edge_pass(row, col, Xs2, Xd2, Xu2, Ea2, N, E):
    C = 128
    H = 64
    mesh = plsc.VectorSubcoreMesh(core_axis_name="c", subcore_axis_name="s",
                                  num_cores=2, num_subcores=16)
    body = functools.partial(_sc_edge_body, N=N, E=E, C=C)
    k = pl.kernel(
        body,
        out_type=(
            jax.ShapeDtypeStruct((2 * E, H), F32),
            jax.ShapeDtypeStruct((2 * N, H), F32),
            jax.ShapeDtypeStruct((2 * N, H), F32),
        ),
        mesh=mesh,
        compiler_params=pltpu.CompilerParams(use_tc_tiling_on_sc=False),
        scratch_types=[
            pltpu.VMEM_SHARED((N, H), F32),
            pltpu.VMEM_SHARED((N, H), F32),
            pltpu.VMEM((C,), jnp.int32),
            pltpu.VMEM((C,), jnp.int32),
            pltpu.VMEM((C,), jnp.int32),
            pltpu.VMEM((C, H), F32),
            pltpu.VMEM((C, H), F32),
            pltpu.VMEM((C, H), F32),
            pltpu.VMEM((C, H), F32),
            pltpu.VMEM((C, H), F32),
            pltpu.SemaphoreType.DMA,
        ],
    )
    return k(row, col, Xs2, Xd2, Xu2, Ea2)


def _stats_body(m_ref, sum_ref, sq_ref, *, blocks_per_half):
    i = pl.program_id(0)

    @pl.when(i % blocks_per_half == 0)
    def _():
        sum_ref[...] = jnp.zeros_like(sum_ref)
        sq_ref[...] = jnp.zeros_like(sq_ref)

    mb = m_ref[...]
    sum_ref[0] += jnp.sum(mb, axis=0, keepdims=True)
    sq_ref[0] += jnp.sum(mb * mb, axis=0, keepdims=True)


def _edge_stats(m2, E, block_rows):
    H = m2.shape[1]
    nblk = (2 * E) // block_rows
    bph = nblk // 2
    body = functools.partial(_stats_body, blocks_per_half=bph)
    return pl.pallas_call(
        body,
        grid=(nblk,),
        in_specs=[pl.BlockSpec((block_rows, H), lambda i: (i, 0))],
        out_specs=[
            pl.BlockSpec((1, 1, H), lambda i, _b=bph: (i // _b, 0, 0)),
            pl.BlockSpec((1, 1, H), lambda i, _b=bph: (i // _b, 0, 0)),
        ],
        out_shape=[
            jax.ShapeDtypeStruct((2, 1, H), F32),
            jax.ShapeDtypeStruct((2, 1, H), F32),
        ],
    )(m2)


def _edge_out_body(mL_ref, mR_ref, ea_ref, sum_ref, sq_ref, g_ref, b_ref,
                   out_ref, *, E):
    mean = sum_ref[...] / E
    var = sq_ref[...] / E - mean * mean
    rstd = lax.rsqrt(var + 1e-5)
    halves = []
    for h, mr in ((0, mL_ref), (1, mR_ref)):
        z = (mr[...] - mean[h]) * (rstd[h] * g_ref[h]) + b_ref[h]
        halves.append(_softplus(z))
    out_ref[...] = ea_ref[...] + jnp.concatenate(halves, axis=1)


def _edge_out(m2, edge_attr, sum2, sq2, g2, b2, E, block_rows):
    H = m2.shape[1]
    n = E // block_rows
    body = functools.partial(_edge_out_body, E=E)
    return pl.pallas_call(
        body,
        grid=(n,),
        in_specs=[
            pl.BlockSpec((block_rows, H), lambda i: (i, 0)),
            pl.BlockSpec((block_rows, H), lambda i, _n=n: (_n + i, 0)),
            pl.BlockSpec((block_rows, 2 * H), lambda i: (i, 0)),
            pl.BlockSpec((2, 1, H), lambda i: (0, 0, 0)),
            pl.BlockSpec((2, 1, H), lambda i: (0, 0, 0)),
            pl.BlockSpec((2, H), lambda i: (0, 0)),
            pl.BlockSpec((2, H), lambda i: (0, 0)),
        ],
        out_specs=pl.BlockSpec((block_rows, 2 * H), lambda i: (i, 0)),
        out_shape=jax.ShapeDtypeStruct((E, 2 * H), F32),
    )(m2, m2, edge_attr, sum2, sq2, g2, b2)


def _node_out_body(x_ref, h2_ref, s2_ref, w_ref, b_ref, g_ref, bb_ref,
                   out_ref):
    hs = jnp.concatenate([h2_ref[0], h2_ref[1]], axis=1)
    ss = jnp.concatenate([s2_ref[0], s2_ref[1]], axis=1)
    hn = hs / (ss + 1e-6)
    xb = x_ref[...]
    u = lax.dot_general(xb, w_ref[...], (((1,), (1,)), ((), ())),
                        preferred_element_type=F32) + b_ref[...] + hn
    mu = jnp.mean(u, axis=0, keepdims=True)
    d = u - mu
    v = jnp.mean(d * d, axis=0, keepdims=True)
    z = d * lax.rsqrt(v + 1e-5) * g_ref[...] + bb_ref[...]
    out_ref[...] = xb + _softplus(z)


def _node_out(x, h2, s2, W_su, b_su, g, b):
    N, D = x.shape
    return pl.pallas_call(
        _node_out_body,
        out_shape=jax.ShapeDtypeStruct((N, D), F32),
    )(x, h2.reshape(2, N, D // 2), s2.reshape(2, N, D // 2),
      W_su, b_su.reshape(1, D), g.reshape(1, D), b.reshape(1, D))


def kernel(x, edge_index, edge_attr, W_sg, b_sg, W_dg, b_dg, W_eg, b_eg,
           W_su, b_su, W_du, b_du, bn_e_g, bn_e_b, bn_n_g, bn_n_b):
    N, D = x.shape
    E = edge_attr.shape[0]
    H = D // 2

    row = edge_index[0]
    col = edge_index[1]

    nb = _pick_block(N, 1000)
    eb = _pick_block(E, 2000)
    Xs2 = _project_halved(x, W_sg, b_sg, nb).reshape(2 * N, H)
    Xd2 = _project_halved(x, W_dg, b_dg, nb).reshape(2 * N, H)
    Xu2 = _project_halved(x, W_du, b_du, nb).reshape(2 * N, H)
    Ea2 = _project_halved(edge_attr, W_eg, b_eg, eb).reshape(2 * E, H)

    m2, h2, s2 = _sc_edge_pass(row, col, Xs2, Xd2, Xu2, Ea2, N, E)

    sum2, sq2 = _edge_stats(m2, E, _pick_block(E, 4000))
    y_new = _edge_out(m2, edge_attr, sum2, sq2,
                      bn_e_g.reshape(2, H), bn_e_b.reshape(2, H), E, eb)
    x_new = _node_out(x, h2, s2, W_su, b_su, bn_n_g, bn_n_b)
    return (x_new, y_new)
